# Initial kernel scaffold; baseline (speedup 1.0000x reference)
import jax
import jax.numpy as jnp
from jax.experimental import pallas as pl


def kernel(x, edge_index, edge_attr, W_gcn, b_gcn, gamma1, beta1, W1, b1, W2, b2, gamma2, beta2):
    raise NotImplementedError("write your pallas kernel here")


# speedup vs baseline: 15.7084x; 15.7084x over previous
import functools

import jax
import jax.numpy as jnp
from jax import lax
from jax.experimental import pallas as pl
from jax.experimental.pallas import tpu as pltpu
from jax.experimental.pallas import tpu_sc as plsc

N = 10000
E = 320000
D = 128
NC = 2
NS = 16
L = 16
NW = NC * NS
EW = E // NW
C = 80
NCHUNK = EW // C
NPAD = 10240
STRIPE = NPAD // NS

_mesh = plsc.VectorSubcoreMesh(core_axis_name="c", subcore_axis_name="s")


@functools.partial(
    pl.kernel,
    mesh=_mesh,
    out_type=jax.ShapeDtypeStruct((NC * NPAD,), jnp.float32),
    scratch_types=[
        pltpu.VMEM((C,), jnp.int32),
        pltpu.VMEM((C,), jnp.float32),
        pltpu.VMEM((STRIPE,), jnp.float32),
        pltpu.VMEM_SHARED((NPAD,), jnp.float32),
    ],
)
def _deg_kernel(dst_hbm, out_hbm, idx_v, ones_v, buf_v, deg_sh):
    c = lax.axis_index("c")
    s = lax.axis_index("s")
    wid = c * NS + s

    for i in range(C // L):
        ones_v[pl.ds(i * L, L)] = jnp.ones((L,), jnp.float32)

    def _zero(i, carry):
        buf_v[pl.ds(i * L, L)] = jnp.zeros((L,), jnp.float32)
        return carry

    lax.fori_loop(0, STRIPE // L, _zero, 0)
    pltpu.sync_copy(buf_v, deg_sh.at[pl.ds(s * STRIPE, STRIPE)])
    plsc.subcore_barrier()

    base0 = wid * EW

    def _body(j, carry):
        pltpu.sync_copy(dst_hbm.at[pl.ds(base0 + j * C, C)], idx_v)
        pltpu.sync_copy(ones_v, deg_sh.at[idx_v], add=True)
        return carry

    lax.fori_loop(0, NCHUNK, _body, 0)
    plsc.subcore_barrier()

    pltpu.sync_copy(deg_sh.at[pl.ds(s * STRIPE, STRIPE)], buf_v)
    pltpu.sync_copy(buf_v, out_hbm.at[pl.ds(c * NPAD + s * STRIPE, STRIPE)])


@functools.partial(
    pl.kernel,
    mesh=_mesh,
    out_type=jax.ShapeDtypeStruct((NC * NPAD, D), jnp.float32),
    scratch_types=[
        pltpu.VMEM((C,), jnp.int32),
        pltpu.VMEM((C,), jnp.int32),
        pltpu.VMEM((C, D), jnp.float32),
        pltpu.VMEM_SHARED((NPAD, D), jnp.float32),
        pltpu.SemaphoreType.DMA,
    ],
)
def _agg_kernel(src_hbm, dst_hbm, y_hbm, out_hbm, sidx_v, didx_v, rows_v,
                agg_sh, sem):
    c = lax.axis_index("c")
    s = lax.axis_index("s")
    wid = c * NS + s

    def _zero(i, carry):
        rows_v[i // (D // L), pl.ds((i % (D // L)) * L, L)] = (
            jnp.zeros((L,), jnp.float32))
        return carry

    lax.fori_loop(0, C * D // L, _zero, 0)
    for k in range(STRIPE // C):
        pltpu.sync_copy(rows_v, agg_sh.at[pl.ds(s * STRIPE + k * C, C)])
    plsc.subcore_barrier()

    base0 = wid * EW

    def _body(j, carry):
        b = base0 + j * C
        pltpu.sync_copy(src_hbm.at[pl.ds(b, C)], sidx_v)
        pltpu.sync_copy(dst_hbm.at[pl.ds(b, C)], didx_v)
        pltpu.async_copy(y_hbm.at[sidx_v], rows_v, sem).wait()
        pltpu.sync_copy(rows_v, agg_sh.at[didx_v], add=True)
        return carry

    lax.fori_loop(0, NCHUNK, _body, 0)
    plsc.subcore_barrier()

    for k in range(STRIPE // C):
        off = s * STRIPE + k * C
        pltpu.sync_copy(agg_sh.at[pl.ds(off, C)], rows_v)
        pltpu.sync_copy(rows_v, out_hbm.at[pl.ds(c * NPAD + off, C)])


def _lin_body(x_ref, w_ref, deg_ref, y_ref, dinv_ref):
    x_lin = jnp.dot(x_ref[...], w_ref[...], preferred_element_type=jnp.float32)
    deg = deg_ref[0:N, :] + deg_ref[NPAD:NPAD + N, :] + 1.0
    dinv = lax.rsqrt(deg)
    dinv_ref[...] = dinv
    y_ref[...] = x_lin * dinv


_lin = pl.pallas_call(
    _lin_body,
    out_shape=(
        jax.ShapeDtypeStruct((N, D), jnp.float32),
        jax.ShapeDtypeStruct((N, 1), jnp.float32),
    ),
)


def _final_body(x_ref, y_ref, agg_ref, dinv_ref, bg_ref, g1_ref, be1_ref,
                w1_ref, b1_ref, w2_ref, b2_ref, g2_ref, be2_ref, out_ref):
    a = agg_ref[0:N, :] + agg_ref[NPAD:NPAD + N, :]
    h_local = (a + y_ref[...]) * dinv_ref[...] + bg_ref[...]
    h = x_ref[...] + h_local
    mean = jnp.mean(h, axis=0)
    var = jnp.mean((h - mean) ** 2, axis=0)
    h = (h - mean) * lax.rsqrt(var + 1e-5) * g1_ref[...] + be1_ref[...]
    ff = jnp.maximum(
        jnp.dot(h, w1_ref[...], preferred_element_type=jnp.float32)
        + b1_ref[...], 0.0)
    ff = jnp.dot(ff, w2_ref[...], preferred_element_type=jnp.float32) + b2_ref[...]
    h2 = h + ff
    mean2 = jnp.mean(h2, axis=0)
    var2 = jnp.mean((h2 - mean2) ** 2, axis=0)
    out_ref[...] = ((h2 - mean2) * lax.rsqrt(var2 + 1e-5) * g2_ref[...]
                    + be2_ref[...])


_final = pl.pallas_call(
    _final_body,
    out_shape=jax.ShapeDtypeStruct((N, D), jnp.float32),
)


def kernel(x, edge_index, edge_attr, W_gcn, b_gcn, gamma1, beta1, W1, b1, W2,
           b2, gamma2, beta2):
    src = edge_index[0]
    dst = edge_index[1]
    deg = _deg_kernel(dst).reshape(NC * NPAD, 1)
    y, dinv = _lin(x, W_gcn, deg)
    agg = _agg_kernel(src, dst, y)
    out = _final(x, y, agg, dinv,
                 b_gcn.reshape(1, D), gamma1.reshape(1, D),
                 beta1.reshape(1, D), W1, b1.reshape(1, 2 * D), W2,
                 b2.reshape(1, D), gamma2.reshape(1, D), beta2.reshape(1, D))
    return out


# speedup vs baseline: 27.6889x; 1.7627x over previous
import functools

import jax
import jax.numpy as jnp
from jax import lax
from jax.experimental import pallas as pl
from jax.experimental.pallas import tpu as pltpu
from jax.experimental.pallas import tpu_sc as plsc

N = 10000
E = 320000
D = 128
NC = 2
NS = 16
L = 16
NW = NC * NS
EW = E // NW
C = 40
NCHUNK = EW // C
NBUF = 5
NITER = NCHUNK // NBUF
NPAD = 10240
STRIPE = NPAD // NS

_mesh = plsc.VectorSubcoreMesh(core_axis_name="c", subcore_axis_name="s")


@functools.partial(
    pl.kernel,
    mesh=_mesh,
    out_type=jax.ShapeDtypeStruct((NC * NPAD,), jnp.float32),
    scratch_types=(
        [pltpu.VMEM((C,), jnp.int32) for _ in range(NBUF)]
        + [pltpu.SemaphoreType.DMA for _ in range(NBUF)]
        + [pltpu.SemaphoreType.DMA for _ in range(NBUF)]
        + [
            pltpu.VMEM((C,), jnp.float32),
            pltpu.VMEM((STRIPE,), jnp.float32),
            pltpu.VMEM_SHARED((NPAD,), jnp.float32),
        ]
    ),
)
def _deg_kernel(dst_hbm, out_hbm, *refs):
    didx = refs[0:NBUF]
    isem = refs[NBUF:2 * NBUF]
    ssem = refs[2 * NBUF:3 * NBUF]
    ones_v, buf_v, deg_sh = refs[3 * NBUF:]
    c = lax.axis_index("c")
    s = lax.axis_index("s")
    wid = c * NS + s
    base0 = wid * EW

    for i in range(C // L):
        ones_v[pl.ds(i * L, L)] = jnp.ones((L,), jnp.float32)

    def _zero(i, carry):
        buf_v[pl.ds(i * L, L)] = jnp.zeros((L,), jnp.float32)
        return carry

    lax.fori_loop(0, STRIPE // L, _zero, 0)
    pltpu.sync_copy(buf_v, deg_sh.at[pl.ds(s * STRIPE, STRIPE)])
    plsc.subcore_barrier()

    def _idx_load(chunk, b):
        pltpu.async_copy(dst_hbm.at[pl.ds(base0 + chunk * C, C)], didx[b],
                         isem[b])

    def _iwait(chunk, b):
        pltpu.make_async_copy(dst_hbm.at[pl.ds(base0 + chunk * C, C)],
                              didx[b], isem[b]).wait()

    def _scatter(b):
        pltpu.async_copy(ones_v, deg_sh.at[didx[b]], ssem[b], add=True)

    def _sdrain(b):
        pltpu.make_async_copy(ones_v, deg_sh.at[didx[b]], ssem[b]).wait()

    for b in range(NBUF):
        _idx_load(b, b)
    for b in range(NBUF):
        _iwait(b, b)
        _scatter(b)

    def _body(jj, carry):
        for b in range(NBUF):
            chunk = jj * NBUF + b
            _sdrain(b)
            _idx_load(chunk, b)
        for b in range(NBUF):
            chunk = jj * NBUF + b
            _iwait(chunk, b)
            _scatter(b)
        return carry

    lax.fori_loop(1, NITER, _body, 0)
    for b in range(NBUF):
        _sdrain(b)
    plsc.subcore_barrier()

    pltpu.sync_copy(deg_sh.at[pl.ds(s * STRIPE, STRIPE)], buf_v)
    pltpu.sync_copy(buf_v, out_hbm.at[pl.ds(c * NPAD + s * STRIPE, STRIPE)])


@functools.partial(
    pl.kernel,
    mesh=_mesh,
    out_type=jax.ShapeDtypeStruct((NC * NPAD, D), jnp.float32),
    scratch_types=(
        [pltpu.VMEM((C, D), jnp.float32) for _ in range(NBUF)]
        + [pltpu.VMEM((C,), jnp.int32) for _ in range(NBUF)]
        + [pltpu.VMEM((C,), jnp.int32) for _ in range(2 * NBUF)]
        + [pltpu.SemaphoreType.DMA for _ in range(NBUF)]
        + [pltpu.SemaphoreType.DMA for _ in range(NBUF)]
        + [pltpu.SemaphoreType.DMA for _ in range(2 * NBUF)]
        + [pltpu.SemaphoreType.DMA for _ in range(NBUF)]
        + [pltpu.VMEM_SHARED((NPAD, D), jnp.float32)]
    ),
)
def _agg_kernel(src_hbm, dst_hbm, y_hbm, out_hbm, *refs):
    rows = refs[0:NBUF]
    didx = refs[NBUF:2 * NBUF]
    sidx = refs[2 * NBUF:4 * NBUF]
    gsem = refs[4 * NBUF:5 * NBUF]
    dsem = refs[5 * NBUF:6 * NBUF]
    ssem_s = refs[6 * NBUF:8 * NBUF]
    ssem = refs[8 * NBUF:9 * NBUF]
    agg_sh = refs[9 * NBUF]
    c = lax.axis_index("c")
    s = lax.axis_index("s")
    wid = c * NS + s
    base0 = wid * EW

    def _zero(i, carry):
        rows[0][i // (D // L), pl.ds((i % (D // L)) * L, L)] = (
            jnp.zeros((L,), jnp.float32))
        return carry

    lax.fori_loop(0, C * D // L, _zero, 0)
    for k in range(STRIPE // C):
        pltpu.sync_copy(rows[0], agg_sh.at[pl.ds(s * STRIPE + k * C, C)])
    plsc.subcore_barrier()

    def _didx_load(chunk, b):
        pltpu.async_copy(dst_hbm.at[pl.ds(base0 + chunk * C, C)], didx[b],
                         dsem[b])

    def _diwait(chunk, b):
        pltpu.make_async_copy(dst_hbm.at[pl.ds(base0 + chunk * C, C)],
                              didx[b], dsem[b]).wait()

    def _sidx_load(chunk, b2):
        pltpu.async_copy(src_hbm.at[pl.ds(base0 + chunk * C, C)], sidx[b2],
                         ssem_s[b2])

    def _siwait(chunk, b2):
        pltpu.make_async_copy(src_hbm.at[pl.ds(base0 + chunk * C, C)],
                              sidx[b2], ssem_s[b2]).wait()

    def _gather(b, b2):
        pltpu.async_copy(y_hbm.at[sidx[b2]], rows[b], gsem[b])

    def _gwait(b, b2):
        pltpu.make_async_copy(y_hbm.at[sidx[b2]], rows[b], gsem[b]).wait()

    def _scatter(b):
        pltpu.async_copy(rows[b], agg_sh.at[didx[b]], ssem[b], add=True)

    def _sdrain(b):
        pltpu.make_async_copy(rows[b], agg_sh.at[didx[b]], ssem[b]).wait()

    for b in range(NBUF):
        _didx_load(b, b)
        _sidx_load(b, b)
        _sidx_load(b + NBUF, b + NBUF)
    for b in range(NBUF):
        _siwait(b, b)
        _gather(b, b)
    for b in range(NBUF):
        _diwait(b, b)
        _gwait(b, b)
        _scatter(b)

    def _round(jj, parity, prefetch):
        po = parity * NBUF
        npo = (1 - parity) * NBUF
        for b in range(NBUF):
            chunk = jj * NBUF + b
            _sdrain(b)
            _didx_load(chunk, b)
            if prefetch:
                _sidx_load(chunk + NBUF, b + npo)
        for b in range(NBUF):
            chunk = jj * NBUF + b
            _siwait(chunk, b + po)
            _gather(b, b + po)
        for b in range(NBUF):
            chunk = jj * NBUF + b
            _diwait(chunk, b)
            _gwait(b, b + po)
            _scatter(b)

    def _body(p, carry):
        _round(2 * p + 1, 1, True)
        _round(2 * p + 2, 0, True)
        return carry

    lax.fori_loop(0, (NITER - 2) // 2, _body, 0)
    _round(NITER - 1, 1, False)
    for b in range(NBUF):
        _sdrain(b)
    plsc.subcore_barrier()

    for k in range(STRIPE // C):
        off = s * STRIPE + k * C
        pltpu.sync_copy(agg_sh.at[pl.ds(off, C)], rows[0])
        pltpu.sync_copy(rows[0], out_hbm.at[pl.ds(c * NPAD + off, C)])


def _lin_body(x_ref, w_ref, deg_ref, y_ref, dinv_ref):
    x_lin = jnp.dot(x_ref[...], w_ref[...], preferred_element_type=jnp.float32)
    deg = deg_ref[0:N, :] + deg_ref[NPAD:NPAD + N, :] + 1.0
    dinv = lax.rsqrt(deg)
    dinv_ref[...] = dinv
    y_ref[...] = x_lin * dinv


_lin = pl.pallas_call(
    _lin_body,
    out_shape=(
        jax.ShapeDtypeStruct((N, D), jnp.float32),
        jax.ShapeDtypeStruct((N, 1), jnp.float32),
    ),
)


def _final_body(x_ref, y_ref, agg_ref, dinv_ref, bg_ref, g1_ref, be1_ref,
                w1_ref, b1_ref, w2_ref, b2_ref, g2_ref, be2_ref, out_ref):
    a = agg_ref[0:N, :] + agg_ref[NPAD:NPAD + N, :]
    h_local = (a + y_ref[...]) * dinv_ref[...] + bg_ref[...]
    h = x_ref[...] + h_local
    mean = jnp.mean(h, axis=0)
    var = jnp.mean((h - mean) ** 2, axis=0)
    h = (h - mean) * lax.rsqrt(var + 1e-5) * g1_ref[...] + be1_ref[...]
    ff = jnp.maximum(
        jnp.dot(h, w1_ref[...], preferred_element_type=jnp.float32)
        + b1_ref[...], 0.0)
    ff = jnp.dot(ff, w2_ref[...], preferred_element_type=jnp.float32) + b2_ref[...]
    h2 = h + ff
    mean2 = jnp.mean(h2, axis=0)
    var2 = jnp.mean((h2 - mean2) ** 2, axis=0)
    out_ref[...] = ((h2 - mean2) * lax.rsqrt(var2 + 1e-5) * g2_ref[...]
                    + be2_ref[...])


_final = pl.pallas_call(
    _final_body,
    out_shape=jax.ShapeDtypeStruct((N, D), jnp.float32),
)


def kernel(x, edge_index, edge_attr, W_gcn, b_gcn, gamma1, beta1, W1, b1, W2,
           b2, gamma2, beta2):
    src = edge_index[0]
    dst = edge_index[1]
    deg = _deg_kernel(dst).reshape(NC * NPAD, 1)
    y, dinv = _lin(x, W_gcn, deg)
    agg = _agg_kernel(src, dst, y)
    out = _final(x, y, agg, dinv,
                 b_gcn.reshape(1, D), gamma1.reshape(1, D),
                 beta1.reshape(1, D), W1, b1.reshape(1, 2 * D), W2,
                 b2.reshape(1, D), gamma2.reshape(1, D), beta2.reshape(1, D))
    return out


# speedup vs baseline: 29.0313x; 1.0485x over previous
import functools

import jax
import jax.numpy as jnp
from jax import lax
from jax.experimental import pallas as pl
from jax.experimental.pallas import tpu as pltpu
from jax.experimental.pallas import tpu_sc as plsc

N = 10000
E = 320000
D = 128
NC = 2
NS = 16
L = 16
NW = NC * NS
EW = E // NW
NPAD = 10240
STRIPE = NPAD // NS

CD = 80
NCHUNKD = EW // CD
NBD = 5
NITERD = NCHUNKD // NBD

CA = 40
NCHUNKA = EW // CA
NBA = 5
NITERA = NCHUNKA // NBA
EPAD = NBA * CA

_mesh = plsc.VectorSubcoreMesh(core_axis_name="c", subcore_axis_name="s")


@functools.partial(
    pl.kernel,
    mesh=_mesh,
    out_type=jax.ShapeDtypeStruct((NC * NPAD,), jnp.float32),
    scratch_types=(
        [pltpu.VMEM((CD,), jnp.int32) for _ in range(NBD)]
        + [pltpu.SemaphoreType.DMA for _ in range(NBD)]
        + [pltpu.SemaphoreType.DMA for _ in range(NBD)]
        + [
            pltpu.VMEM((CD,), jnp.float32),
            pltpu.VMEM((STRIPE,), jnp.float32),
            pltpu.VMEM_SHARED((NPAD,), jnp.float32),
        ]
    ),
)
def _deg_kernel(dst_hbm, out_hbm, *refs):
    didx = refs[0:NBD]
    isem = refs[NBD:2 * NBD]
    ssem = refs[2 * NBD:3 * NBD]
    ones_v, buf_v, deg_sh = refs[3 * NBD:]
    c = lax.axis_index("c")
    s = lax.axis_index("s")
    wid = c * NS + s
    base0 = wid * EW

    for i in range(CD // L):
        ones_v[pl.ds(i * L, L)] = jnp.ones((L,), jnp.float32)

    def _zero(i, carry):
        buf_v[pl.ds(i * L, L)] = jnp.zeros((L,), jnp.float32)
        return carry

    lax.fori_loop(0, STRIPE // L, _zero, 0)
    pltpu.sync_copy(buf_v, deg_sh.at[pl.ds(s * STRIPE, STRIPE)])
    plsc.subcore_barrier()

    def _idx_load(chunk, b):
        pltpu.async_copy(dst_hbm.at[pl.ds(base0 + chunk * CD, CD)], didx[b],
                         isem[b])

    def _iwait(chunk, b):
        pltpu.make_async_copy(dst_hbm.at[pl.ds(base0 + chunk * CD, CD)],
                              didx[b], isem[b]).wait()

    def _scatter(b):
        pltpu.async_copy(ones_v, deg_sh.at[didx[b]], ssem[b], add=True)

    def _sdrain(b):
        pltpu.make_async_copy(ones_v, deg_sh.at[didx[b]], ssem[b]).wait()

    for b in range(NBD):
        _idx_load(b, b)
    for b in range(NBD):
        _iwait(b, b)
        _scatter(b)

    def _body(jj, carry):
        for b in range(NBD):
            chunk = jj * NBD + b
            _sdrain(b)
            _idx_load(chunk, b)
        for b in range(NBD):
            chunk = jj * NBD + b
            _iwait(chunk, b)
            _scatter(b)
        return carry

    lax.fori_loop(1, NITERD, _body, 0)
    for b in range(NBD):
        _sdrain(b)
    plsc.subcore_barrier()

    pltpu.sync_copy(deg_sh.at[pl.ds(s * STRIPE, STRIPE)], buf_v)
    pltpu.sync_copy(buf_v, out_hbm.at[pl.ds(c * NPAD + s * STRIPE, STRIPE)])


@functools.partial(
    pl.kernel,
    mesh=_mesh,
    out_type=jax.ShapeDtypeStruct((NC * NPAD, D), jnp.float32),
    scratch_types=(
        [pltpu.VMEM((CA, D), jnp.float32) for _ in range(NBA)]
        + [pltpu.VMEM((CA,), jnp.int32) for _ in range(NBA)]
        + [pltpu.VMEM((CA,), jnp.int32) for _ in range(2 * NBA)]
        + [pltpu.SemaphoreType.DMA for _ in range(NBA)]
        + [pltpu.SemaphoreType.DMA for _ in range(NBA)]
        + [pltpu.SemaphoreType.DMA for _ in range(2 * NBA)]
        + [pltpu.SemaphoreType.DMA for _ in range(NBA)]
        + [pltpu.VMEM_SHARED((NPAD, D), jnp.float32)]
    ),
)
def _agg_kernel(src_hbm, dst_hbm, y_hbm, out_hbm, *refs):
    rows = refs[0:NBA]
    didx = refs[NBA:2 * NBA]
    sidx = refs[2 * NBA:4 * NBA]
    gsem = refs[4 * NBA:5 * NBA]
    dsem = refs[5 * NBA:6 * NBA]
    ssem_s = refs[6 * NBA:8 * NBA]
    ssem = refs[8 * NBA:9 * NBA]
    agg_sh = refs[9 * NBA]
    c = lax.axis_index("c")
    s = lax.axis_index("s")
    wid = c * NS + s
    base0 = wid * EW

    def _zero(i, carry):
        rows[0][i // (D // L), pl.ds((i % (D // L)) * L, L)] = (
            jnp.zeros((L,), jnp.float32))
        return carry

    lax.fori_loop(0, CA * D // L, _zero, 0)
    for k in range(STRIPE // CA):
        pltpu.sync_copy(rows[0], agg_sh.at[pl.ds(s * STRIPE + k * CA, CA)])
    plsc.subcore_barrier()

    def _didx_load(chunk, b):
        pltpu.async_copy(dst_hbm.at[pl.ds(base0 + chunk * CA, CA)], didx[b],
                         dsem[b])

    def _diwait(chunk, b):
        pltpu.make_async_copy(dst_hbm.at[pl.ds(base0 + chunk * CA, CA)],
                              didx[b], dsem[b]).wait()

    def _sidx_load(chunk, b2):
        pltpu.async_copy(src_hbm.at[pl.ds(base0 + chunk * CA, CA)], sidx[b2],
                         ssem_s[b2])

    def _siwait(chunk, b2):
        pltpu.make_async_copy(src_hbm.at[pl.ds(base0 + chunk * CA, CA)],
                              sidx[b2], ssem_s[b2]).wait()

    def _gather(b, b2):
        pltpu.async_copy(y_hbm.at[sidx[b2]], rows[b], gsem[b])

    def _gwait(b, b2):
        pltpu.make_async_copy(y_hbm.at[sidx[b2]], rows[b], gsem[b]).wait()

    def _scatter(b):
        pltpu.async_copy(rows[b], agg_sh.at[didx[b]], ssem[b], add=True)

    def _sdrain(b):
        pltpu.make_async_copy(rows[b], agg_sh.at[didx[b]], ssem[b]).wait()

    for b in range(NBA):
        _didx_load(b, b)
        _sidx_load(b, b)
        _sidx_load(b + NBA, b + NBA)
    for b in range(NBA):
        _siwait(b, b)
        _gather(b, b)
    for b in range(NBA):
        _diwait(b, b)
        _gwait(b, b)
        _scatter(b)

    def _round(jj, parity, prefetch):
        po = parity * NBA
        npo = (1 - parity) * NBA
        for b in range(NBA):
            chunk = jj * NBA + b
            _sdrain(b)
            _didx_load(chunk, b)
            if prefetch:
                _sidx_load(chunk + NBA, b + npo)
        for b in range(NBA):
            chunk = jj * NBA + b
            _siwait(chunk, b + po)
            _gather(b, b + po)
        for b in range(NBA):
            chunk = jj * NBA + b
            _diwait(chunk, b)
            _gwait(b, b + po)
            _scatter(b)

    def _body(p, carry):
        _round(2 * p + 1, 1, True)
        _round(2 * p + 2, 0, True)
        return carry

    lax.fori_loop(0, (NITERA - 2) // 2, _body, 0)
    _round(NITERA - 1, 1, False)
    for b in range(NBA):
        _sdrain(b)
    plsc.subcore_barrier()

    for k in range(STRIPE // CA):
        off = s * STRIPE + k * CA
        pltpu.sync_copy(agg_sh.at[pl.ds(off, CA)], rows[0])
        pltpu.sync_copy(rows[0], out_hbm.at[pl.ds(c * NPAD + off, CA)])


def _lin_body(x_ref, w_ref, deg_ref, y_ref, dinv_ref):
    x_lin = jnp.dot(x_ref[...], w_ref[...], preferred_element_type=jnp.float32)
    deg = deg_ref[0:N, :] + deg_ref[NPAD:NPAD + N, :] + 1.0
    dinv = lax.rsqrt(deg)
    dinv_ref[...] = dinv
    y_ref[...] = x_lin * dinv


_lin = pl.pallas_call(
    _lin_body,
    out_shape=(
        jax.ShapeDtypeStruct((N, D), jnp.float32),
        jax.ShapeDtypeStruct((N, 1), jnp.float32),
    ),
)


def _final_body(x_ref, y_ref, agg_ref, dinv_ref, bg_ref, g1_ref, be1_ref,
                w1_ref, b1_ref, w2_ref, b2_ref, g2_ref, be2_ref, out_ref):
    a = agg_ref[0:N, :] + agg_ref[NPAD:NPAD + N, :]
    h_local = (a + y_ref[...]) * dinv_ref[...] + bg_ref[...]
    h = x_ref[...] + h_local
    mean = jnp.mean(h, axis=0)
    var = jnp.mean((h - mean) ** 2, axis=0)
    h = (h - mean) * lax.rsqrt(var + 1e-5) * g1_ref[...] + be1_ref[...]
    ff = jnp.maximum(
        jnp.dot(h, w1_ref[...], preferred_element_type=jnp.float32)
        + b1_ref[...], 0.0)
    ff = jnp.dot(ff, w2_ref[...], preferred_element_type=jnp.float32) + b2_ref[...]
    h2 = h + ff
    mean2 = jnp.mean(h2, axis=0)
    var2 = jnp.mean((h2 - mean2) ** 2, axis=0)
    out_ref[...] = ((h2 - mean2) * lax.rsqrt(var2 + 1e-5) * g2_ref[...]
                    + be2_ref[...])


_final = pl.pallas_call(
    _final_body,
    out_shape=jax.ShapeDtypeStruct((N, D), jnp.float32),
)


def kernel(x, edge_index, edge_attr, W_gcn, b_gcn, gamma1, beta1, W1, b1, W2,
           b2, gamma2, beta2):
    src = edge_index[0]
    dst = edge_index[1]
    pad = jnp.zeros((EPAD,), jnp.int32)
    src_p = jnp.concatenate([src, pad])
    dst_p = jnp.concatenate([dst, pad])
    deg = _deg_kernel(dst).reshape(NC * NPAD, 1)
    y, dinv = _lin(x, W_gcn, deg)
    agg = _agg_kernel(src_p, dst_p, y)
    out = _final(x, y, agg, dinv,
                 b_gcn.reshape(1, D), gamma1.reshape(1, D),
                 beta1.reshape(1, D), W1, b1.reshape(1, 2 * D), W2,
                 b2.reshape(1, D), gamma2.reshape(1, D), beta2.reshape(1, D))
    return out


# speedup vs baseline: 30.2641x; 1.0425x over previous
import functools

import jax
import jax.numpy as jnp
from jax import lax
from jax.experimental import pallas as pl
from jax.experimental.pallas import tpu as pltpu
from jax.experimental.pallas import tpu_sc as plsc

N = 10000
E = 320000
D = 128
NC = 2
NS = 16
L = 16
NW = NC * NS
EW = E // NW
NPAD = 10240
STRIPE = NPAD // NS

CD = 80
NCHUNKD = EW // CD
NBD = 5
NITERD = NCHUNKD // NBD

CA = 80
NCHUNKA = EW // CA
NBA = 4
NFULL = (NCHUNKA - 1) // NBA

_mesh = plsc.VectorSubcoreMesh(core_axis_name="c", subcore_axis_name="s")


@functools.partial(
    pl.kernel,
    mesh=_mesh,
    out_type=jax.ShapeDtypeStruct((NC * NPAD,), jnp.float32),
    scratch_types=(
        [pltpu.VMEM((CD,), jnp.int32) for _ in range(NBD)]
        + [pltpu.SemaphoreType.DMA for _ in range(NBD)]
        + [pltpu.SemaphoreType.DMA for _ in range(NBD)]
        + [
            pltpu.VMEM((CD,), jnp.float32),
            pltpu.VMEM((STRIPE,), jnp.float32),
            pltpu.VMEM_SHARED((NPAD,), jnp.float32),
        ]
    ),
)
def _deg_kernel(dst_hbm, out_hbm, *refs):
    didx = refs[0:NBD]
    isem = refs[NBD:2 * NBD]
    ssem = refs[2 * NBD:3 * NBD]
    ones_v, buf_v, deg_sh = refs[3 * NBD:]
    c = lax.axis_index("c")
    s = lax.axis_index("s")
    wid = c * NS + s
    base0 = wid * EW

    for i in range(CD // L):
        ones_v[pl.ds(i * L, L)] = jnp.ones((L,), jnp.float32)

    def _zero(i, carry):
        buf_v[pl.ds(i * L, L)] = jnp.zeros((L,), jnp.float32)
        return carry

    lax.fori_loop(0, STRIPE // L, _zero, 0)
    pltpu.sync_copy(buf_v, deg_sh.at[pl.ds(s * STRIPE, STRIPE)])
    plsc.subcore_barrier()

    def _idx_load(chunk, b):
        pltpu.async_copy(dst_hbm.at[pl.ds(base0 + chunk * CD, CD)], didx[b],
                         isem[b])

    def _iwait(chunk, b):
        pltpu.make_async_copy(dst_hbm.at[pl.ds(base0 + chunk * CD, CD)],
                              didx[b], isem[b]).wait()

    def _scatter(b):
        pltpu.async_copy(ones_v, deg_sh.at[didx[b]], ssem[b], add=True)

    def _sdrain(b):
        pltpu.make_async_copy(ones_v, deg_sh.at[didx[b]], ssem[b]).wait()

    for b in range(NBD):
        _idx_load(b, b)
    for b in range(NBD):
        _iwait(b, b)
        _scatter(b)

    def _body(jj, carry):
        for b in range(NBD):
            chunk = jj * NBD + b
            _sdrain(b)
            _idx_load(chunk, b)
        for b in range(NBD):
            chunk = jj * NBD + b
            _iwait(chunk, b)
            _scatter(b)
        return carry

    lax.fori_loop(1, NITERD, _body, 0)
    for b in range(NBD):
        _sdrain(b)
    plsc.subcore_barrier()

    pltpu.sync_copy(deg_sh.at[pl.ds(s * STRIPE, STRIPE)], buf_v)
    pltpu.sync_copy(buf_v, out_hbm.at[pl.ds(c * NPAD + s * STRIPE, STRIPE)])


@functools.partial(
    pl.kernel,
    mesh=_mesh,
    out_type=jax.ShapeDtypeStruct((NC * NPAD, D), jnp.float32),
    scratch_types=(
        [pltpu.VMEM((CA, D), jnp.float32) for _ in range(NBA)]
        + [pltpu.VMEM((CA,), jnp.int32) for _ in range(NBA)]
        + [pltpu.VMEM((CA,), jnp.int32) for _ in range(2 * NBA)]
        + [pltpu.SemaphoreType.DMA for _ in range(NBA)]
        + [pltpu.SemaphoreType.DMA for _ in range(NBA)]
        + [pltpu.SemaphoreType.DMA for _ in range(2 * NBA)]
        + [pltpu.SemaphoreType.DMA for _ in range(NBA)]
        + [pltpu.VMEM_SHARED((NPAD, D), jnp.float32)]
    ),
)
def _agg_kernel(src_hbm, dst_hbm, y_hbm, out_hbm, *refs):
    rows = refs[0:NBA]
    didx = refs[NBA:2 * NBA]
    sidx = refs[2 * NBA:4 * NBA]
    gsem = refs[4 * NBA:5 * NBA]
    dsem = refs[5 * NBA:6 * NBA]
    ssem_s = refs[6 * NBA:8 * NBA]
    ssem = refs[8 * NBA:9 * NBA]
    agg_sh = refs[9 * NBA]
    c = lax.axis_index("c")
    s = lax.axis_index("s")
    wid = c * NS + s
    base0 = wid * EW

    def _zero(i, carry):
        rows[0][i // (D // L), pl.ds((i % (D // L)) * L, L)] = (
            jnp.zeros((L,), jnp.float32))
        return carry

    lax.fori_loop(0, CA * D // L, _zero, 0)
    for k in range(STRIPE // CA):
        pltpu.sync_copy(rows[0], agg_sh.at[pl.ds(s * STRIPE + k * CA, CA)])
    plsc.subcore_barrier()

    def _didx_load(chunk, b):
        pltpu.async_copy(dst_hbm.at[pl.ds(base0 + chunk * CA, CA)], didx[b],
                         dsem[b])

    def _diwait(chunk, b):
        pltpu.make_async_copy(dst_hbm.at[pl.ds(base0 + chunk * CA, CA)],
                              didx[b], dsem[b]).wait()

    def _sidx_load(chunk, b2):
        pltpu.async_copy(src_hbm.at[pl.ds(base0 + chunk * CA, CA)], sidx[b2],
                         ssem_s[b2])

    def _siwait(chunk, b2):
        pltpu.make_async_copy(src_hbm.at[pl.ds(base0 + chunk * CA, CA)],
                              sidx[b2], ssem_s[b2]).wait()

    def _gather(b, b2):
        pltpu.async_copy(y_hbm.at[sidx[b2]], rows[b], gsem[b])

    def _gwait(b, b2):
        pltpu.make_async_copy(y_hbm.at[sidx[b2]], rows[b], gsem[b]).wait()

    def _scatter(b):
        pltpu.async_copy(rows[b], agg_sh.at[didx[b]], ssem[b], add=True)

    def _sdrain(b):
        pltpu.make_async_copy(rows[b], agg_sh.at[didx[b]], ssem[b]).wait()

    tchunk = NCHUNKA - 1
    _didx_load(tchunk, 0)
    _sidx_load(tchunk, 0)
    _siwait(tchunk, 0)
    _gather(0, 0)
    _diwait(tchunk, 0)
    _gwait(0, 0)
    _scatter(0)
    _sdrain(0)

    for b in range(NBA):
        _didx_load(b, b)
        _sidx_load(b, b)
        _sidx_load(b + NBA, b + NBA)
    for b in range(NBA):
        _siwait(b, b)
        _gather(b, b)
    for b in range(NBA):
        _diwait(b, b)
        _gwait(b, b)
        _scatter(b)

    def _round(jj, parity, prefetch):
        po = parity * NBA
        npo = (1 - parity) * NBA
        for b in range(NBA):
            chunk = jj * NBA + b
            _sdrain(b)
            _didx_load(chunk, b)
            if prefetch:
                _sidx_load(chunk + NBA, b + npo)
        for b in range(NBA):
            chunk = jj * NBA + b
            _siwait(chunk, b + po)
            _gather(b, b + po)
        for b in range(NBA):
            chunk = jj * NBA + b
            _diwait(chunk, b)
            _gwait(b, b + po)
            _scatter(b)

    def _body(p, carry):
        _round(2 * p + 1, 1, True)
        _round(2 * p + 2, 0, True)
        return carry

    lax.fori_loop(0, (NFULL - 3) // 2, _body, 0)
    _round(NFULL - 2, 1, True)
    _round(NFULL - 1, 0, False)
    for b in range(NBA):
        _sdrain(b)
    plsc.subcore_barrier()

    for k in range(STRIPE // CA):
        off = s * STRIPE + k * CA
        pltpu.sync_copy(agg_sh.at[pl.ds(off, CA)], rows[0])
        pltpu.sync_copy(rows[0], out_hbm.at[pl.ds(c * NPAD + off, CA)])


def _lin_body(x_ref, w_ref, deg_ref, y_ref, dinv_ref):
    x_lin = jnp.dot(x_ref[...], w_ref[...], preferred_element_type=jnp.float32)
    deg = deg_ref[0:N, :] + deg_ref[NPAD:NPAD + N, :] + 1.0
    dinv = lax.rsqrt(deg)
    dinv_ref[...] = dinv
    y_ref[...] = x_lin * dinv


_lin = pl.pallas_call(
    _lin_body,
    out_shape=(
        jax.ShapeDtypeStruct((N, D), jnp.float32),
        jax.ShapeDtypeStruct((N, 1), jnp.float32),
    ),
)


def _final_body(x_ref, y_ref, agg_ref, dinv_ref, bg_ref, g1_ref, be1_ref,
                w1_ref, b1_ref, w2_ref, b2_ref, g2_ref, be2_ref, out_ref):
    a = agg_ref[0:N, :] + agg_ref[NPAD:NPAD + N, :]
    h_local = (a + y_ref[...]) * dinv_ref[...] + bg_ref[...]
    h = x_ref[...] + h_local
    mean = jnp.mean(h, axis=0)
    var = jnp.mean((h - mean) ** 2, axis=0)
    h = (h - mean) * lax.rsqrt(var + 1e-5) * g1_ref[...] + be1_ref[...]
    ff = jnp.maximum(
        jnp.dot(h, w1_ref[...], preferred_element_type=jnp.float32)
        + b1_ref[...], 0.0)
    ff = jnp.dot(ff, w2_ref[...], preferred_element_type=jnp.float32) + b2_ref[...]
    h2 = h + ff
    mean2 = jnp.mean(h2, axis=0)
    var2 = jnp.mean((h2 - mean2) ** 2, axis=0)
    out_ref[...] = ((h2 - mean2) * lax.rsqrt(var2 + 1e-5) * g2_ref[...]
                    + be2_ref[...])


_final = pl.pallas_call(
    _final_body,
    out_shape=jax.ShapeDtypeStruct((N, D), jnp.float32),
)


def kernel(x, edge_index, edge_attr, W_gcn, b_gcn, gamma1, beta1, W1, b1, W2,
           b2, gamma2, beta2):
    src = edge_index[0]
    dst = edge_index[1]
    deg = _deg_kernel(dst).reshape(NC * NPAD, 1)
    y, dinv = _lin(x, W_gcn, deg)
    agg = _agg_kernel(src, dst, y)
    out = _final(x, y, agg, dinv,
                 b_gcn.reshape(1, D), gamma1.reshape(1, D),
                 beta1.reshape(1, D), W1, b1.reshape(1, 2 * D), W2,
                 b2.reshape(1, D), gamma2.reshape(1, D), beta2.reshape(1, D))
    return out


# speedup vs baseline: 30.2813x; 1.0006x over previous
import functools

import jax
import jax.numpy as jnp
from jax import lax
from jax.experimental import pallas as pl
from jax.experimental.pallas import tpu as pltpu
from jax.experimental.pallas import tpu_sc as plsc

N = 10000
E = 320000
D = 128
NC = 2
NS = 16
L = 16
NW = NC * NS
EW = E // NW
NPAD = 10240
STRIPE = NPAD // NS

CD = 80
NCHUNKD = EW // CD
NBD = 5
NITERD = NCHUNKD // NBD

CA = 80
NCHUNKA = EW // CA
NBA = 4
NTAIL = NCHUNKA - 31 * NBA
NFULL = (NCHUNKA - NTAIL) // NBA

_mesh = plsc.VectorSubcoreMesh(core_axis_name="c", subcore_axis_name="s")


@functools.partial(
    pl.kernel,
    mesh=_mesh,
    out_type=jax.ShapeDtypeStruct((NC * NPAD,), jnp.float32),
    scratch_types=(
        [pltpu.VMEM((CD,), jnp.int32) for _ in range(NBD)]
        + [pltpu.SemaphoreType.DMA for _ in range(NBD)]
        + [pltpu.SemaphoreType.DMA for _ in range(NBD)]
        + [
            pltpu.VMEM((CD,), jnp.float32),
            pltpu.VMEM((STRIPE,), jnp.float32),
            pltpu.VMEM_SHARED((NPAD,), jnp.float32),
        ]
    ),
)
def _deg_kernel(dst_hbm, out_hbm, *refs):
    didx = refs[0:NBD]
    isem = refs[NBD:2 * NBD]
    ssem = refs[2 * NBD:3 * NBD]
    ones_v, buf_v, deg_sh = refs[3 * NBD:]
    c = lax.axis_index("c")
    s = lax.axis_index("s")
    wid = c * NS + s
    base0 = wid * EW

    for i in range(CD // L):
        ones_v[pl.ds(i * L, L)] = jnp.ones((L,), jnp.float32)

    def _zero(i, carry):
        buf_v[pl.ds(i * L, L)] = jnp.zeros((L,), jnp.float32)
        return carry

    lax.fori_loop(0, STRIPE // L, _zero, 0)
    pltpu.sync_copy(buf_v, deg_sh.at[pl.ds(s * STRIPE, STRIPE)])
    plsc.subcore_barrier()

    def _idx_load(chunk, b):
        pltpu.async_copy(dst_hbm.at[pl.ds(base0 + chunk * CD, CD)], didx[b],
                         isem[b])

    def _iwait(chunk, b):
        pltpu.make_async_copy(dst_hbm.at[pl.ds(base0 + chunk * CD, CD)],
                              didx[b], isem[b]).wait()

    def _scatter(b):
        pltpu.async_copy(ones_v, deg_sh.at[didx[b]], ssem[b], add=True)

    def _sdrain(b):
        pltpu.make_async_copy(ones_v, deg_sh.at[didx[b]], ssem[b]).wait()

    for b in range(NBD):
        _idx_load(b, b)
    for b in range(NBD):
        _iwait(b, b)
        _scatter(b)

    def _body(jj, carry):
        for b in range(NBD):
            chunk = jj * NBD + b
            _sdrain(b)
            _idx_load(chunk, b)
        for b in range(NBD):
            chunk = jj * NBD + b
            _iwait(chunk, b)
            _scatter(b)
        return carry

    lax.fori_loop(1, NITERD, _body, 0)
    for b in range(NBD):
        _sdrain(b)
    plsc.subcore_barrier()

    pltpu.sync_copy(deg_sh.at[pl.ds(s * STRIPE, STRIPE)], buf_v)
    pltpu.sync_copy(buf_v, out_hbm.at[pl.ds(c * NPAD + s * STRIPE, STRIPE)])


@functools.partial(
    pl.kernel,
    mesh=_mesh,
    out_type=jax.ShapeDtypeStruct((NC * NPAD, D), jnp.float32),
    scratch_types=(
        [pltpu.VMEM((CA, D), jnp.float32) for _ in range(NBA)]
        + [pltpu.VMEM((CA,), jnp.int32) for _ in range(NBA)]
        + [pltpu.VMEM((CA,), jnp.int32) for _ in range(2 * NBA)]
        + [pltpu.SemaphoreType.DMA for _ in range(NBA)]
        + [pltpu.SemaphoreType.DMA for _ in range(NBA)]
        + [pltpu.SemaphoreType.DMA for _ in range(2 * NBA)]
        + [pltpu.SemaphoreType.DMA for _ in range(NBA)]
        + [pltpu.VMEM_SHARED((NPAD, D), jnp.float32)]
    ),
)
def _agg_kernel(src_hbm, dst_hbm, y_hbm, out_hbm, *refs):
    rows = refs[0:NBA]
    didx = refs[NBA:2 * NBA]
    sidx = refs[2 * NBA:4 * NBA]
    gsem = refs[4 * NBA:5 * NBA]
    dsem = refs[5 * NBA:6 * NBA]
    ssem_s = refs[6 * NBA:8 * NBA]
    ssem = refs[8 * NBA:9 * NBA]
    agg_sh = refs[9 * NBA]
    c = lax.axis_index("c")
    s = lax.axis_index("s")
    wid = c * NS + s
    base0 = wid * EW

    def _zero(i, carry):
        rows[0][i // (D // L), pl.ds((i % (D // L)) * L, L)] = (
            jnp.zeros((L,), jnp.float32))
        return carry

    lax.fori_loop(0, CA * D // L, _zero, 0)
    for k in range(STRIPE // CA):
        pltpu.sync_copy(rows[0], agg_sh.at[pl.ds(s * STRIPE + k * CA, CA)])
    plsc.subcore_barrier()

    def _didx_load(chunk, b):
        pltpu.async_copy(dst_hbm.at[pl.ds(base0 + chunk * CA, CA)], didx[b],
                         dsem[b])

    def _diwait(chunk, b):
        pltpu.make_async_copy(dst_hbm.at[pl.ds(base0 + chunk * CA, CA)],
                              didx[b], dsem[b]).wait()

    def _sidx_load(chunk, b2):
        pltpu.async_copy(src_hbm.at[pl.ds(base0 + chunk * CA, CA)], sidx[b2],
                         ssem_s[b2])

    def _siwait(chunk, b2):
        pltpu.make_async_copy(src_hbm.at[pl.ds(base0 + chunk * CA, CA)],
                              sidx[b2], ssem_s[b2]).wait()

    def _gather(b, b2):
        pltpu.async_copy(y_hbm.at[sidx[b2]], rows[b], gsem[b])

    def _gwait(b, b2):
        pltpu.make_async_copy(y_hbm.at[sidx[b2]], rows[b], gsem[b]).wait()

    def _scatter(b):
        pltpu.async_copy(rows[b], agg_sh.at[didx[b]], ssem[b], add=True)

    def _sdrain(b):
        pltpu.make_async_copy(rows[b], agg_sh.at[didx[b]], ssem[b]).wait()

    for t in range(NTAIL):
        tchunk = NCHUNKA - NTAIL + t
        _didx_load(tchunk, 0)
        _sidx_load(tchunk, 0)
        _siwait(tchunk, 0)
        _gather(0, 0)
        _diwait(tchunk, 0)
        _gwait(0, 0)
        _scatter(0)
        _sdrain(0)

    for b in range(NBA):
        _didx_load(b, b)
        _sidx_load(b, b)
        _sidx_load(b + NBA, b + NBA)
    for b in range(NBA):
        _siwait(b, b)
        _gather(b, b)
    for b in range(NBA):
        _diwait(b, b)
        _gwait(b, b)
        _scatter(b)

    def _round(jj, parity, prefetch):
        po = parity * NBA
        npo = (1 - parity) * NBA
        for b in range(NBA):
            chunk = jj * NBA + b
            _sdrain(b)
            _didx_load(chunk, b)
            if prefetch:
                _sidx_load(chunk + NBA, b + npo)
        for b in range(NBA):
            chunk = jj * NBA + b
            _siwait(chunk, b + po)
            _gather(b, b + po)
        for b in range(NBA):
            chunk = jj * NBA + b
            _diwait(chunk, b)
            _gwait(b, b + po)
            _scatter(b)

    def _body(p, carry):
        _round(2 * p + 1, 1, True)
        _round(2 * p + 2, 0, True)
        return carry

    lax.fori_loop(0, (NFULL - 3) // 2, _body, 0)
    _round(NFULL - 2, 1, True)
    _round(NFULL - 1, 0, False)
    for b in range(NBA):
        _sdrain(b)
    plsc.subcore_barrier()

    for k in range(STRIPE // CA):
        off = s * STRIPE + k * CA
        pltpu.sync_copy(agg_sh.at[pl.ds(off, CA)], rows[0])
        pltpu.sync_copy(rows[0], out_hbm.at[pl.ds(c * NPAD + off, CA)])


def _lin_body(x_ref, w_ref, deg_ref, y_ref, dinv_ref):
    x_lin = jnp.dot(x_ref[...], w_ref[...], preferred_element_type=jnp.float32)
    deg = deg_ref[0:N, :] + deg_ref[NPAD:NPAD + N, :] + 1.0
    dinv = lax.rsqrt(deg)
    dinv_ref[...] = dinv
    y_ref[...] = x_lin * dinv


_lin = pl.pallas_call(
    _lin_body,
    out_shape=(
        jax.ShapeDtypeStruct((N, D), jnp.float32),
        jax.ShapeDtypeStruct((N, 1), jnp.float32),
    ),
)


def _final_body(x_ref, y_ref, agg_ref, dinv_ref, bg_ref, g1_ref, be1_ref,
                w1_ref, b1_ref, w2_ref, b2_ref, g2_ref, be2_ref, out_ref):
    a = agg_ref[0:N, :] + agg_ref[NPAD:NPAD + N, :]
    h_local = (a + y_ref[...]) * dinv_ref[...] + bg_ref[...]
    h = x_ref[...] + h_local
    mean = jnp.mean(h, axis=0)
    var = jnp.mean((h - mean) ** 2, axis=0)
    h = (h - mean) * lax.rsqrt(var + 1e-5) * g1_ref[...] + be1_ref[...]
    ff = jnp.maximum(
        jnp.dot(h, w1_ref[...], preferred_element_type=jnp.float32)
        + b1_ref[...], 0.0)
    ff = jnp.dot(ff, w2_ref[...], preferred_element_type=jnp.float32) + b2_ref[...]
    h2 = h + ff
    mean2 = jnp.mean(h2, axis=0)
    var2 = jnp.mean((h2 - mean2) ** 2, axis=0)
    out_ref[...] = ((h2 - mean2) * lax.rsqrt(var2 + 1e-5) * g2_ref[...]
                    + be2_ref[...])


_final = pl.pallas_call(
    _final_body,
    out_shape=jax.ShapeDtypeStruct((N, D), jnp.float32),
)


def kernel(x, edge_index, edge_attr, W_gcn, b_gcn, gamma1, beta1, W1, b1, W2,
           b2, gamma2, beta2):
    src = edge_index[0]
    dst = edge_index[1]
    deg = _deg_kernel(dst).reshape(NC * NPAD, 1)
    y, dinv = _lin(x, W_gcn, deg)
    agg = _agg_kernel(src, dst, y)
    out = _final(x, y, agg, dinv,
                 b_gcn.reshape(1, D), gamma1.reshape(1, D),
                 beta1.reshape(1, D), W1, b1.reshape(1, 2 * D), W2,
                 b2.reshape(1, D), gamma2.reshape(1, D), beta2.reshape(1, D))
    return out
